# Initial kernel scaffold; baseline (speedup 1.0000x reference)
import jax
import jax.numpy as jnp
from jax.experimental import pallas as pl


def kernel(x, W1, W2, b2):
    raise NotImplementedError("write your pallas kernel here")


# speedup vs baseline: 7.4145x; 7.4145x over previous
import jax
import jax.numpy as jnp
from jax import lax
from jax.experimental import pallas as pl

B = 16
IN_SIZE = 256
OUT_SIZE = 256
N = 1024
NUM_UNITS = 8
MED_IDX = N - 1 - (N - 1) // 2
CHUNK = 256

_DOT = dict(preferred_element_type=jnp.float32,
            precision=jax.lax.Precision.HIGHEST)


def _stats_kernel(x_ref, w1_ref, stats_ref):
    x = x_ref[0]
    w1 = w1_ref[...]
    u = lax.dot_general(w1, x, (((1,), (0,)), ((), ())),
                        preferred_element_type=jnp.float32)
    a_row = jnp.sqrt(jnp.sum(u * u, axis=0, keepdims=True))
    s_row = jnp.sqrt(jnp.sum(x * x, axis=0, keepdims=True))
    stats_ref[0, 0:1, :] = s_row

    eye = (lax.broadcasted_iota(jnp.int32, (N, N), 0) ==
           lax.broadcasted_iota(jnp.int32, (N, N), 1)).astype(jnp.float32)
    a_col = lax.dot_general(eye, a_row, (((1,), (1,)), ((), ())), **_DOT)

    for c in range(N // CHUNK):
        a_sub = a_row[:, c * CHUNK:(c + 1) * CHUNK]
        gt = a_col > a_sub
        eq = a_col == a_sub
        m_idx = lax.broadcasted_iota(jnp.int32, (N, CHUNK), 0)
        n_idx = lax.broadcasted_iota(jnp.int32, (N, CHUNK), 1) + c * CHUNK
        cnt = jnp.where(gt | (eq & (m_idx < n_idx)), 1.0, 0.0)
        stats_ref[0, 1:2, c * CHUNK:(c + 1) * CHUNK] = (
            jnp.sum(cnt, axis=0, keepdims=True))

    rank_row = stats_ref[0, 1:2, :]
    amax = jnp.max(a_row)
    amed = jnp.sum(jnp.where(rank_row == float(MED_IDX), a_row, 0.0))
    stats_ref[0, 2:3, :] = jnp.broadcast_to(amed / amax, (1, N))
    stats_ref[0, 3:8, :] = jnp.zeros((5, N), jnp.float32)


def _combine_kernel(stats_ref, x_ref, w1_ref, out_ref):
    b = pl.program_id(0)
    ratios = stats_ref[:, 2:3, 0:1]
    prop = jnp.sum(ratios) / B
    k_f = jnp.floor(prop * N)

    own = stats_ref[pl.ds(b, 1), :, :]
    s_row = own[0, 0:1, :]
    rank_row = own[0, 1:2, :]
    w = jnp.where(rank_row < k_f, s_row, 0.0)
    wn = w / jnp.sum(w)

    x = x_ref[0]
    w1 = w1_ref[...]
    y = lax.dot_general(wn, x, (((1,), (1,)), ((), ())), **_DOT)
    pose = lax.dot_general(y, w1, (((1,), (1,)), ((), ())), **_DOT)
    out_ref[0] = jnp.broadcast_to(pose, (NUM_UNITS, OUT_SIZE))


def kernel(x, W1, W2, b2):
    del W2, b2
    w1 = W1.reshape(OUT_SIZE, IN_SIZE)

    stats = pl.pallas_call(
        _stats_kernel,
        grid=(B,),
        in_specs=[
            pl.BlockSpec((1, IN_SIZE, N), lambda b: (b, 0, 0)),
            pl.BlockSpec((OUT_SIZE, IN_SIZE), lambda b: (0, 0)),
        ],
        out_specs=pl.BlockSpec((1, 8, N), lambda b: (b, 0, 0)),
        out_shape=jax.ShapeDtypeStruct((B, 8, N), jnp.float32),
    )(x, w1)

    pose = pl.pallas_call(
        _combine_kernel,
        grid=(B,),
        in_specs=[
            pl.BlockSpec((B, 8, N), lambda b: (0, 0, 0)),
            pl.BlockSpec((1, IN_SIZE, N), lambda b: (b, 0, 0)),
            pl.BlockSpec((OUT_SIZE, IN_SIZE), lambda b: (0, 0)),
        ],
        out_specs=pl.BlockSpec((1, NUM_UNITS, OUT_SIZE), lambda b: (b, 0, 0)),
        out_shape=jax.ShapeDtypeStruct((B, NUM_UNITS, OUT_SIZE), jnp.float32),
    )(stats, x, w1)
    return pose
